# Initial kernel scaffold; baseline (speedup 1.0000x reference)
import jax
import jax.numpy as jnp
from jax.experimental import pallas as pl


def kernel(x, edge_index, params):
    raise NotImplementedError("write your pallas kernel here")


# speedup vs baseline: 1041.8434x; 1041.8434x over previous
import functools
import math

import jax
import jax.numpy as jnp
from jax import lax
from jax.experimental import pallas as pl
from jax.experimental.pallas import tpu as pltpu
from jax.experimental.pallas import tpu_sc as plsc

_N_HEADS = 8
_EMBED = 128
_FF = 512
_KEY_DIM = _EMBED // _N_HEADS


def _indeg_counts(dst, n_nodes):
    (e,) = dst.shape
    n_sub = 16
    e_w = e // n_sub
    assert e % (n_sub * 16) == 0 and n_nodes % 16 == 0

    mesh = plsc.VectorSubcoreMesh(core_axis_name="c", subcore_axis_name="s")

    @functools.partial(
        pl.kernel,
        out_type=jax.ShapeDtypeStruct((n_nodes,), jnp.float32),
        mesh=mesh,
        scratch_types=[
            pltpu.VMEM((e_w,), jnp.int32),
            pltpu.VMEM((e_w,), jnp.float32),
            pltpu.VMEM((n_nodes,), jnp.float32),
            pltpu.VMEM_SHARED((n_nodes,), jnp.float32),
        ],
    )
    def indeg_kernel(dst_hbm, out_hbm, idx_v, ones_v, zeros_v, acc_shared):
        c = lax.axis_index("c")
        s = lax.axis_index("s")

        def fill_ones(i, carry):
            ones_v[pl.ds(i * 16, 16)] = jnp.full((16,), 1.0, jnp.float32)
            return carry

        lax.fori_loop(0, e_w // 16, fill_ones, 0)

        @pl.when(s == 0)
        def _():
            def fill_zeros(i, carry):
                zeros_v[pl.ds(i * 16, 16)] = jnp.zeros((16,), jnp.float32)
                return carry

            lax.fori_loop(0, n_nodes // 16, fill_zeros, 0)
            pltpu.sync_copy(zeros_v, acc_shared)

        plsc.subcore_barrier()
        pltpu.sync_copy(dst_hbm.at[pl.ds(s * e_w, e_w)], idx_v)
        pltpu.sync_copy(ones_v, acc_shared.at[idx_v], add=True)
        plsc.subcore_barrier()

        @pl.when((s == 0) & (c == 0))
        def _():
            pltpu.sync_copy(acc_shared, out_hbm)

    return indeg_kernel(dst)


def _dense_body(x_ref, cnt_ref, wv_ref, wo_ref, bn1w_ref, bn1b_ref,
                w1_ref, b1_ref, w2_ref, b2_ref, bn2w_ref, bn2b_ref, out_ref):
    h = x_ref[...]
    mask = jnp.where(cnt_ref[...] > 0.0, 1.0, 0.0)
    n_layers = wv_ref.shape[0]
    for l in range(n_layers):
        v = jnp.dot(h, wv_ref[l])
        g = h + jnp.dot(v * mask, wo_ref[l])
        m = jnp.mean(g, axis=0, keepdims=True)
        d = g - m
        var = jnp.mean(d * d, axis=0, keepdims=True)
        h = d / jnp.sqrt(var + 1e-5) * bn1w_ref[l] + bn1b_ref[l]
        t = jnp.maximum(jnp.dot(h, w1_ref[l]) + b1_ref[l], 0.0)
        g2 = h + (jnp.dot(t, w2_ref[l]) + b2_ref[l])
        m2 = jnp.mean(g2, axis=0, keepdims=True)
        d2 = g2 - m2
        var2 = jnp.mean(d2 * d2, axis=0, keepdims=True)
        h = d2 / jnp.sqrt(var2 + 1e-5) * bn2w_ref[l] + bn2b_ref[l]
    out_ref[...] = h


def kernel(x, edge_index, params):
    b, g, d = x.shape
    n = b * g
    dst = edge_index[1]

    counts = _indeg_counts(dst, n)

    wv = jnp.stack([p['Wv'].transpose(1, 0, 2).reshape(_EMBED, _N_HEADS * _KEY_DIM)
                    for p in params])
    wo = jnp.stack([p['Wo'].reshape(_N_HEADS * _KEY_DIM, _EMBED) for p in params])
    bn1w = jnp.stack([p['bn1_w'].reshape(1, _EMBED) for p in params])
    bn1b = jnp.stack([p['bn1_b'].reshape(1, _EMBED) for p in params])
    w1 = jnp.stack([p['ff_w1'] for p in params])
    b1 = jnp.stack([p['ff_b1'].reshape(1, _FF) for p in params])
    w2 = jnp.stack([p['ff_w2'] for p in params])
    b2 = jnp.stack([p['ff_b2'].reshape(1, _EMBED) for p in params])
    bn2w = jnp.stack([p['bn2_w'].reshape(1, _EMBED) for p in params])
    bn2b = jnp.stack([p['bn2_b'].reshape(1, _EMBED) for p in params])

    out = pl.pallas_call(
        _dense_body,
        out_shape=jax.ShapeDtypeStruct((n, d), jnp.float32),
    )(x.reshape(n, d), counts.reshape(n, 1),
      wv, wo, bn1w, bn1b, w1, b1, w2, b2, bn2w, bn2b)
    return out.reshape(b, g, d)


# speedup vs baseline: 1085.8441x; 1.0422x over previous
import functools
import math

import jax
import jax.numpy as jnp
from jax import lax
from jax.experimental import pallas as pl
from jax.experimental.pallas import tpu as pltpu
from jax.experimental.pallas import tpu_sc as plsc

_N_HEADS = 8
_EMBED = 128
_FF = 512
_KEY_DIM = _EMBED // _N_HEADS


def _indeg_counts(dst, n_nodes):
    (e,) = dst.shape
    n_sub = 16
    e_w = e // (2 * n_sub)
    assert e % (2 * n_sub * 16) == 0 and n_nodes % 16 == 0

    mesh = plsc.VectorSubcoreMesh(core_axis_name="c", subcore_axis_name="s")

    @functools.partial(
        pl.kernel,
        out_type=jax.ShapeDtypeStruct((2, n_nodes), jnp.float32),
        mesh=mesh,
        scratch_types=[
            pltpu.VMEM((e_w,), jnp.int32),
            pltpu.VMEM((e_w,), jnp.float32),
            pltpu.VMEM((n_nodes,), jnp.float32),
            pltpu.VMEM_SHARED((n_nodes,), jnp.float32),
        ],
    )
    def indeg_kernel(dst_hbm, out_hbm, idx_v, ones_v, zeros_v, acc_shared):
        c = lax.axis_index("c")
        s = lax.axis_index("s")

        def fill_ones(i, carry):
            ones_v[pl.ds(i * 16, 16)] = jnp.full((16,), 1.0, jnp.float32)
            return carry

        lax.fori_loop(0, e_w // 16, fill_ones, 0)

        @pl.when(s == 0)
        def _():
            def fill_zeros(i, carry):
                zeros_v[pl.ds(i * 16, 16)] = jnp.zeros((16,), jnp.float32)
                return carry

            lax.fori_loop(0, n_nodes // 16, fill_zeros, 0)
            pltpu.sync_copy(zeros_v, acc_shared)

        plsc.subcore_barrier()
        pltpu.sync_copy(dst_hbm.at[pl.ds((c * n_sub + s) * e_w, e_w)], idx_v)
        pltpu.sync_copy(ones_v, acc_shared.at[idx_v], add=True)
        plsc.subcore_barrier()

        @pl.when(s == 0)
        def _():
            pltpu.sync_copy(acc_shared, out_hbm.at[c])

    return indeg_kernel(dst)


def _dense_body(x_ref, cnt0_ref, cnt1_ref,
                wv0_ref, wo0_ref, bn1w0_ref, bn1b0_ref, w10_ref, b10_ref,
                w20_ref, b20_ref, bn2w0_ref, bn2b0_ref,
                wv1_ref, wo1_ref, bn1w1_ref, bn1b1_ref, w11_ref, b11_ref,
                w21_ref, b21_ref, bn2w1_ref, bn2b1_ref, out_ref):
    h = x_ref[...]
    mask = jnp.where(cnt0_ref[...] + cnt1_ref[...] > 0.0, 1.0, 0.0)
    layers = (
        (wv0_ref, wo0_ref, bn1w0_ref, bn1b0_ref, w10_ref, b10_ref,
         w20_ref, b20_ref, bn2w0_ref, bn2b0_ref),
        (wv1_ref, wo1_ref, bn1w1_ref, bn1b1_ref, w11_ref, b11_ref,
         w21_ref, b21_ref, bn2w1_ref, bn2b1_ref),
    )
    for (wv, wo, bn1w, bn1b, w1, b1, w2, b2, bn2w, bn2b) in layers:
        v = jnp.dot(h, wv[...])
        g = h + jnp.dot(v * mask, wo[...])
        m = jnp.mean(g, axis=0, keepdims=True)
        d = g - m
        var = jnp.mean(d * d, axis=0, keepdims=True)
        h = d / jnp.sqrt(var + 1e-5) * bn1w[...] + bn1b[...]
        t = jnp.maximum(jnp.dot(h, w1[...]) + b1[...], 0.0)
        g2 = h + (jnp.dot(t, w2[...]) + b2[...])
        m2 = jnp.mean(g2, axis=0, keepdims=True)
        d2 = g2 - m2
        var2 = jnp.mean(d2 * d2, axis=0, keepdims=True)
        h = d2 / jnp.sqrt(var2 + 1e-5) * bn2w[...] + bn2b[...]
    out_ref[...] = h


def kernel(x, edge_index, params):
    b, g, d = x.shape
    n = b * g
    dst = edge_index[1]

    counts = _indeg_counts(dst, n)

    def layer_args(p):
        return (
            p['Wv'].transpose(1, 0, 2).reshape(_EMBED, _N_HEADS * _KEY_DIM),
            p['Wo'].reshape(_N_HEADS * _KEY_DIM, _EMBED),
            p['bn1_w'].reshape(1, _EMBED), p['bn1_b'].reshape(1, _EMBED),
            p['ff_w1'], p['ff_b1'].reshape(1, _FF),
            p['ff_w2'], p['ff_b2'].reshape(1, _EMBED),
            p['bn2_w'].reshape(1, _EMBED), p['bn2_b'].reshape(1, _EMBED),
        )

    out = pl.pallas_call(
        _dense_body,
        out_shape=jax.ShapeDtypeStruct((n, d), jnp.float32),
    )(x.reshape(n, d), counts[0].reshape(n, 1), counts[1].reshape(n, 1),
      *layer_args(params[0]), *layer_args(params[1]))
    return out.reshape(b, g, d)


# speedup vs baseline: 1272.6593x; 1.1720x over previous
import functools
import math

import jax
import jax.numpy as jnp
from jax import lax
from jax.experimental import pallas as pl
from jax.experimental.pallas import tpu as pltpu
from jax.experimental.pallas import tpu_sc as plsc

_N_HEADS = 8
_EMBED = 128
_FF = 512
_KEY_DIM = _EMBED // _N_HEADS


def _indeg_counts(dst, n_nodes):
    (e,) = dst.shape
    n_sub = 16
    e_w = e // (2 * n_sub)
    assert e % (2 * n_sub * 16) == 0 and n_nodes % 80 == 0

    mesh = plsc.VectorSubcoreMesh(core_axis_name="c", subcore_axis_name="s")

    @functools.partial(
        pl.kernel,
        out_type=jax.ShapeDtypeStruct((2, n_nodes), jnp.float32),
        mesh=mesh,
        scratch_types=[
            pltpu.VMEM((e_w,), jnp.int32),
            pltpu.VMEM((e_w,), jnp.float32),
            pltpu.VMEM((n_nodes,), jnp.float32),
            pltpu.VMEM_SHARED((n_nodes,), jnp.float32),
        ],
    )
    def indeg_kernel(dst_hbm, out_hbm, idx_v, ones_v, zeros_v, acc_shared):
        c = lax.axis_index("c")
        s = lax.axis_index("s")
        n_slice = n_nodes // 10
        n_fill = ((n_slice + 15) // 16) * 16

        def fill_ones(i, carry):
            ones_v[pl.ds(i * 16, 16)] = jnp.full((16,), 1.0, jnp.float32)
            return carry

        lax.fori_loop(0, e_w // 16, fill_ones, 0)

        def fill_zeros(i, carry):
            zeros_v[pl.ds(i * 16, 16)] = jnp.zeros((16,), jnp.float32)
            return carry

        lax.fori_loop(0, n_fill // 16, fill_zeros, 0)

        @pl.when(s < 10)
        def _():
            pltpu.sync_copy(zeros_v.at[pl.ds(0, n_slice)],
                            acc_shared.at[pl.ds(s * n_slice, n_slice)])

        plsc.subcore_barrier()
        pltpu.sync_copy(dst_hbm.at[pl.ds((c * n_sub + s) * e_w, e_w)], idx_v)
        pltpu.sync_copy(ones_v, acc_shared.at[idx_v], add=True)
        plsc.subcore_barrier()

        @pl.when(s == 0)
        def _():
            pltpu.sync_copy(acc_shared, out_hbm.at[c])

    return indeg_kernel(dst)


def _mm(a, b):
    return jax.lax.dot(a, b, preferred_element_type=jnp.float32)


def _dense_body(x_ref, cnt_ref,
                wv0_ref, wo0_ref, bn1w0_ref, bn1b0_ref, w10_ref, b10_ref,
                w20_ref, b20_ref, bn2w0_ref, bn2b0_ref,
                wv1_ref, wo1_ref, bn1w1_ref, bn1b1_ref, w11_ref, b11_ref,
                w21_ref, b21_ref, bn2w1_ref, bn2b1_ref, out_ref):
    h = x_ref[...]
    tot = jax.lax.dot_general(cnt_ref[...], jnp.ones((2, 1), jnp.float32),
                              (((0,), (0,)), ((), ())))
    mask = jnp.where(tot > 0.0, 1.0, 0.0)
    layers = (
        (wv0_ref, wo0_ref, bn1w0_ref, bn1b0_ref, w10_ref, b10_ref,
         w20_ref, b20_ref, bn2w0_ref, bn2b0_ref),
        (wv1_ref, wo1_ref, bn1w1_ref, bn1b1_ref, w11_ref, b11_ref,
         w21_ref, b21_ref, bn2w1_ref, bn2b1_ref),
    )
    for (wv, wo, bn1w, bn1b, w1, b1, w2, b2, bn2w, bn2b) in layers:
        wv_flat = jnp.concatenate(
            [wv[pl.ds(hh * _EMBED, _EMBED), :] for hh in range(_N_HEADS)],
            axis=1)
        v = _mm(h, wv_flat)
        g = h + _mm(v * mask, wo[...])
        m = jnp.mean(g, axis=0, keepdims=True)
        d = g - m
        var = jnp.mean(d * d, axis=0, keepdims=True)
        h = d / jnp.sqrt(var + 1e-5) * bn1w[...] + bn1b[...]
        t = jnp.maximum(_mm(h, w1[...]) + b1[...], 0.0)
        g2 = h + (_mm(t, w2[...]) + b2[...])
        m2 = jnp.mean(g2, axis=0, keepdims=True)
        d2 = g2 - m2
        var2 = jnp.mean(d2 * d2, axis=0, keepdims=True)
        h = d2 / jnp.sqrt(var2 + 1e-5) * bn2w[...] + bn2b[...]
    out_ref[...] = h


def kernel(x, edge_index, params):
    b, g, d = x.shape
    n = b * g
    dst = edge_index[1]

    counts = _indeg_counts(dst, n)

    def layer_args(p):
        return (
            p['Wv'].reshape(_N_HEADS * _EMBED, _KEY_DIM),
            p['Wo'].reshape(_N_HEADS * _KEY_DIM, _EMBED),
            p['bn1_w'].reshape(1, _EMBED), p['bn1_b'].reshape(1, _EMBED),
            p['ff_w1'], p['ff_b1'].reshape(1, _FF),
            p['ff_w2'], p['ff_b2'].reshape(1, _EMBED),
            p['bn2_w'].reshape(1, _EMBED), p['bn2_b'].reshape(1, _EMBED),
        )

    out = pl.pallas_call(
        _dense_body,
        out_shape=jax.ShapeDtypeStruct((n, d), jnp.float32),
    )(x.reshape(n, d), counts,
      *layer_args(params[0]), *layer_args(params[1]))
    return out.reshape(b, g, d)


# speedup vs baseline: 1545.5974x; 1.2145x over previous
import functools
import math

import jax
import jax.numpy as jnp
from jax import lax
from jax.experimental import pallas as pl
from jax.experimental.pallas import tpu as pltpu
from jax.experimental.pallas import tpu_sc as plsc

_N_HEADS = 8
_EMBED = 128
_FF = 512
_KEY_DIM = _EMBED // _N_HEADS


def _indeg_counts(edge_index, n_nodes):
    _, e = edge_index.shape
    assert e % 128 == 0 and n_nodes % 80 == 0
    n_chunks = e // 128
    n_w = max(w for w in range(1, 33) if n_chunks % w == 0)
    e_w = e // n_w

    mesh = plsc.VectorSubcoreMesh(core_axis_name="c", subcore_axis_name="s")

    @functools.partial(
        pl.kernel,
        out_type=jax.ShapeDtypeStruct((2, n_nodes), jnp.float32),
        mesh=mesh,
        scratch_types=[
            pltpu.VMEM((2, e_w), jnp.int32),
            pltpu.VMEM((e_w,), jnp.int32),
            pltpu.VMEM((e_w,), jnp.float32),
            pltpu.VMEM((n_nodes,), jnp.float32),
            pltpu.VMEM_SHARED((n_nodes,), jnp.float32),
        ],
    )
    def indeg_kernel(ei_hbm, out_hbm, blk_v, idx_v, ones_v, zeros_v,
                     acc_shared):
        c = lax.axis_index("c")
        s = lax.axis_index("s")
        rank = 2 * s + c

        def fill_ones(i, carry):
            ones_v[pl.ds(i * 16, 16)] = jnp.full((16,), 1.0, jnp.float32)
            return carry

        lax.fori_loop(0, e_w // 16, fill_ones, 0)

        n_slice = n_nodes // 10
        n_fill = ((n_slice + 15) // 16) * 16

        def fill_zeros(i, carry):
            zeros_v[pl.ds(i * 16, 16)] = jnp.zeros((16,), jnp.float32)
            return carry

        lax.fori_loop(0, n_fill // 16, fill_zeros, 0)

        @pl.when(s < 10)
        def _():
            pltpu.sync_copy(zeros_v.at[pl.ds(0, n_slice)],
                            acc_shared.at[pl.ds(s * n_slice, n_slice)])

        plsc.subcore_barrier()

        @pl.when(rank < n_w)
        def _():
            pltpu.sync_copy(ei_hbm.at[:, pl.ds(rank * e_w, e_w)], blk_v)

            def copy_row(i, carry):
                idx_v[pl.ds(i * 16, 16)] = blk_v[1, pl.ds(i * 16, 16)]
                return carry

            lax.fori_loop(0, e_w // 16, copy_row, 0)
            pltpu.sync_copy(ones_v, acc_shared.at[idx_v], add=True)

        plsc.subcore_barrier()

        @pl.when(s == 0)
        def _():
            pltpu.sync_copy(acc_shared, out_hbm.at[c])

    return indeg_kernel(edge_index)


def _mm(a, b):
    return jax.lax.dot(a, b, preferred_element_type=jnp.float32)


def _dense_body(x_ref, cnt_ref,
                wv0_ref, wo0_ref, bn1w0_ref, bn1b0_ref, w10_ref, b10_ref,
                w20_ref, b20_ref, bn2w0_ref, bn2b0_ref,
                wv1_ref, wo1_ref, bn1w1_ref, bn1b1_ref, w11_ref, b11_ref,
                w21_ref, b21_ref, bn2w1_ref, bn2b1_ref, out_ref):
    h = x_ref[...]
    tot = jax.lax.dot_general(cnt_ref[...], jnp.ones((2, 1), jnp.float32),
                              (((0,), (0,)), ((), ())))
    mask = jnp.where(tot > 0.0, 1.0, 0.0)
    layers = (
        (wv0_ref, wo0_ref, bn1w0_ref, bn1b0_ref, w10_ref, b10_ref,
         w20_ref, b20_ref, bn2w0_ref, bn2b0_ref),
        (wv1_ref, wo1_ref, bn1w1_ref, bn1b1_ref, w11_ref, b11_ref,
         w21_ref, b21_ref, bn2w1_ref, bn2b1_ref),
    )
    for (wv, wo, bn1w, bn1b, w1, b1, w2, b2, bn2w, bn2b) in layers:
        wv_flat = jnp.concatenate(
            [wv[pl.ds(hh * _EMBED, _EMBED), :] for hh in range(_N_HEADS)],
            axis=1)
        v = _mm(h, wv_flat)
        g = h + _mm(v * mask, wo[...])

        def bn(g, w, b):
            m = jnp.mean(g, axis=0, keepdims=True)
            msq = jnp.mean(g * g, axis=0, keepdims=True)
            scale = w / jnp.sqrt(msq - m * m + 1e-5)
            return g * scale + (b - m * scale)

        h = bn(g, bn1w[...], bn1b[...])
        t = jnp.maximum(_mm(h, w1[...]) + b1[...], 0.0)
        g2 = h + (_mm(t, w2[...]) + b2[...])
        h = bn(g2, bn2w[...], bn2b[...])
    out_ref[...] = h


def kernel(x, edge_index, params):
    b, g, d = x.shape
    n = b * g

    counts = _indeg_counts(edge_index, n)

    def layer_args(p):
        return (
            p['Wv'].reshape(_N_HEADS * _EMBED, _KEY_DIM),
            p['Wo'].reshape(_N_HEADS * _KEY_DIM, _EMBED),
            p['bn1_w'].reshape(1, _EMBED), p['bn1_b'].reshape(1, _EMBED),
            p['ff_w1'], p['ff_b1'].reshape(1, _FF),
            p['ff_w2'], p['ff_b2'].reshape(1, _EMBED),
            p['bn2_w'].reshape(1, _EMBED), p['bn2_b'].reshape(1, _EMBED),
        )

    out = pl.pallas_call(
        _dense_body,
        out_shape=jax.ShapeDtypeStruct((n, d), jnp.float32),
    )(x.reshape(n, d), counts,
      *layer_args(params[0]), *layer_args(params[1]))
    return out.reshape(b, g, d)
